# baseline (speedup 1.0000x reference)
import functools

import jax
import jax.numpy as jnp
from jax.experimental import pallas as pl
from jax.experimental.pallas import tpu as pltpu

B, M, D = 256, 128, 8
FA, FB, FP = 64, 16, 64
RADIUS, T_STEPS = 3, 2
MD = M * D
BM = 4

_PREC = jax.lax.Precision.DEFAULT


def _dot(a, b):
    return jax.lax.dot(a, b, precision=_PREC, preferred_element_type=jnp.float32)


def _dot_x(sel, x):
    xh = x.astype(jnp.bfloat16).astype(jnp.float32)
    return _dot(sel, xh) + _dot(sel, x - xh)


def _lrelu(x):
    return jnp.where(x >= 0, x, 0.01 * x)


def _elu(x):
    return jnp.where(x > 0, x, jnp.exp(jnp.minimum(x, 0.0)) - 1.0)


def _gru(x, h, Wi_t, Wh_t, bi, bh):
    gi = _dot(x, Wi_t) + bi
    gh = _dot(h, Wh_t) + bh
    r = jax.nn.sigmoid(gi[:, :FP] + gh[:, :FP])
    z = jax.nn.sigmoid(gi[:, FP:2 * FP] + gh[:, FP:2 * FP])
    n = jnp.tanh(gi[:, 2 * FP:] + r * gh[:, 2 * FP:])
    return (1.0 - z) * n + z * h


def _body(atom_ref, bond_ref, adeg_ref, bdeg_ref, amask_ref,
          Wa_t, ba, WnA_t, WnB_t, bn,
          cat0, catR, bal, bt, Wi_t, Wh_t, bi, bh,
          wma_x_rep, wma_m_rep, bma, Wmt_t, bmt, mWi_t, mWh_t, mbi, mbh,
          wo, bo,
          af_out_ref, pred_out_ref):
    atom = atom_ref[...].reshape(BM * M, FA)
    bond = bond_ref[...].reshape(BM * M, FB)
    adeg = adeg_ref[...]
    bdeg = bdeg_ref[...]
    amask = amask_ref[...].reshape(BM * M, 1)

    col = jax.lax.broadcasted_iota(jnp.int32, (MD, M), 1)
    ohA = [(adeg[i] == col).astype(jnp.float32) for i in range(BM)]
    ohB = [(bdeg[i] == col).astype(jnp.float32) for i in range(BM)]
    seg = jax.lax.broadcasted_iota(jnp.int32, (M, MD), 1) // D
    S = (seg == jax.lax.broadcasted_iota(jnp.int32, (M, MD), 0)).astype(jnp.float32)

    adeg_f = adeg.reshape(BM * MD, 1)
    smask = jnp.where(jnp.broadcast_to(adeg_f, (BM * MD, FP)) == M - 1, -9e8, 0.0)

    atom_feat = _lrelu(_dot(atom, Wa_t[...]) + ba[...])
    P = _dot(atom, WnA_t[...])
    Q = _dot(bond, WnB_t[...])
    nf = _lrelu(jnp.concatenate(
        [_dot_x(ohA[i], P[i * M:(i + 1) * M]) + _dot_x(ohB[i], Q[i * M:(i + 1) * M])
         for i in range(BM)], axis=0) + bn[...])

    h = atom_feat
    for r in range(RADIUS):
        if r == 0:
            gath = _dot(nf, cat0[...])
            sa = _dot(atom_feat, catR[0, :, 2 * FP:])
        else:
            af = jnp.maximum(h, 0.0)
            af_ext = _dot(af, catR[r])
            gath = jnp.concatenate(
                [_dot_x(ohA[i], af_ext[i * M:(i + 1) * M, :2 * FP]) for i in range(BM)],
                axis=0)
            sa = af_ext[:, 2 * FP:]
        att = gath[:, :FP] + bt[r]
        sn = gath[:, FP:]
        score = (sn.reshape(BM * M, D, FP) + sa[:, None, :]).reshape(BM * MD, FP)
        score = _lrelu(score + bal[r]) + smask
        e = jnp.exp(score)
        X = jnp.concatenate([e * att, e], axis=1)
        nd = jnp.concatenate(
            [_dot_x(S, X[i * MD:(i + 1) * MD]) for i in range(BM)], axis=0)
        ctx = _elu(nd[:, :FP] / jnp.maximum(nd[:, FP:], 1e-30))
        x_h = atom_feat if r == 0 else h
        h = _gru(ctx, x_h, Wi_t[r], Wh_t[r], bi[r], bh[r])

    af_out_ref[...] = h.reshape(BM, M, FP)
    af_fin = jnp.maximum(h, 0.0)

    amask64 = jnp.broadcast_to(amask, (BM * M, FP))
    msm = jnp.where(amask64 == 0.0, -9e8, 0.0)
    mseg = jax.lax.broadcasted_iota(jnp.int32, (BM, BM * M), 1) // M
    onesb = (mseg == jax.lax.broadcasted_iota(jnp.int32, (BM, BM * M), 0)).astype(jnp.float32)

    mol = _dot_x(onesb, af_fin * amask64)
    afm = jnp.maximum(mol, 0.0)
    attm = _dot(af_fin, Wmt_t[...]) + bmt[...]
    sx = _dot(af_fin, wma_x_rep[...])
    for _t in range(T_STEPS):
        sm = _dot(afm, wma_m_rep[...])
        score = (sx.reshape(BM, M, FP) + sm[:, None, :]).reshape(BM * M, FP)
        score = _lrelu(score + bma[...]) + msm
        e = jnp.exp(score)
        nd = _dot_x(onesb, jnp.concatenate([e * attm, e], axis=1))
        mctx = _elu(nd[:, :FP] / jnp.maximum(nd[:, FP:], 1e-30))
        mol = _gru(mctx, mol, mWi_t[...], mWh_t[...], mbi[...], mbh[...])
        afm = jnp.maximum(mol, 0.0)

    pred = jnp.sum(mol * wo[...], axis=1, keepdims=True) + bo[...]
    pred_out_ref[...] = jnp.broadcast_to(pred.reshape(1, BM, 1), (1, BM, 128))


def kernel(atom_list, bond_list, atom_degree_list, bond_degree_list, atom_mask, params):
    p = params
    Wa, ba = p['atom_fc']
    Wn, bn = p['neighbor_fc']
    Wma, bma = p['mol_align']
    Wmt, bmt = p['mol_attend']
    mWi, mWh, mbi, mbh = p['mol_gru']
    Wo, bo = p['output']

    def rep(v):
        return jnp.broadcast_to(v[:, None], (FP, FP))

    cat0 = jnp.concatenate([p['attend'][0][0].T, rep(p['align'][0][0][0, FP:])], axis=1)
    catR = jnp.stack([
        jnp.concatenate([p['attend'][r][0].T,
                         rep(p['align'][r][0][0, FP:]),
                         rep(p['align'][r][0][0, :FP])], axis=1)
        for r in range(RADIUS)])
    bal = jnp.stack([p['align'][r][1] for r in range(RADIUS)])[:, :, None]
    bt = jnp.stack([p['attend'][r][1] for r in range(RADIUS)])[:, None, :]
    Wi_t = jnp.stack([p['gru'][r][0].T for r in range(RADIUS)])
    Wh_t = jnp.stack([p['gru'][r][1].T for r in range(RADIUS)])
    bi = jnp.stack([p['gru'][r][2] for r in range(RADIUS)])[:, None, :]
    bh = jnp.stack([p['gru'][r][3] for r in range(RADIUS)])[:, None, :]

    adeg = atom_degree_list.astype(jnp.int32).reshape(B, MD, 1)
    bdeg = bond_degree_list.astype(jnp.int32).reshape(B, MD, 1)
    amask = atom_mask.reshape(B, M, 1)

    data_in = [atom_list, bond_list, adeg, bdeg, amask]
    data_specs = [
        pl.BlockSpec((BM, M, FA), lambda i: (i, 0, 0)),
        pl.BlockSpec((BM, M, FB), lambda i: (i, 0, 0)),
        pl.BlockSpec((BM, MD, 1), lambda i: (i, 0, 0)),
        pl.BlockSpec((BM, MD, 1), lambda i: (i, 0, 0)),
        pl.BlockSpec((BM, M, 1), lambda i: (i, 0, 0)),
    ]
    param_in = [
        Wa.T, ba[None], Wn[:, :FA].T, Wn[:, FA:].T, bn[None],
        cat0, catR, bal, bt, Wi_t, Wh_t, bi, bh,
        rep(Wma[0, FP:]), rep(Wma[0, :FP]), bma[None], Wmt.T, bmt[None],
        mWi.T, mWh.T, mbi[None], mbh[None],
        Wo, bo[None],
    ]
    param_specs = [
        pl.BlockSpec(x.shape, functools.partial(lambda n, i: (0,) * n, x.ndim))
        for x in param_in
    ]

    out_shapes = [
        jax.ShapeDtypeStruct((B, M, FP), jnp.float32),
        jax.ShapeDtypeStruct((B // BM, BM, 128), jnp.float32),
    ]
    out_specs = [
        pl.BlockSpec((BM, M, FP), lambda i: (i, 0, 0)),
        pl.BlockSpec((1, BM, 128), lambda i: (i, 0, 0)),
    ]

    af, pred = pl.pallas_call(
        _body,
        grid=(B // BM,),
        in_specs=data_specs + param_specs,
        out_specs=out_specs,
        out_shape=out_shapes,
        compiler_params=pltpu.CompilerParams(
            dimension_semantics=("arbitrary",),
        ),
    )(*data_in, *param_in)
    return (af, pred.reshape(B, 128)[:, :1])


# speedup vs baseline: 1.0007x; 1.0007x over previous
import functools

import jax
import jax.numpy as jnp
from jax.experimental import pallas as pl
from jax.experimental.pallas import tpu as pltpu

B, M, D = 256, 128, 8
FA, FB, FP = 64, 16, 64
RADIUS, T_STEPS = 3, 2
MD = M * D
BM = 4

_PREC = jax.lax.Precision.DEFAULT


def _dot(a, b):
    return jax.lax.dot(a, b, precision=_PREC, preferred_element_type=jnp.float32)


def _dot_h(a, b):
    return jax.lax.dot(a, b, precision=jax.lax.Precision.HIGHEST,
                       preferred_element_type=jnp.float32)


def _dot_x(sel, x):
    xh = x.astype(jnp.bfloat16).astype(jnp.float32)
    return _dot(sel, xh) + _dot(sel, x - xh)


def _lrelu(x):
    return jnp.where(x >= 0, x, 0.01 * x)


def _elu(x):
    return jnp.where(x > 0, x, jnp.exp(jnp.minimum(x, 0.0)) - 1.0)


def _gru(x, h, Wi_t, Wh_t, bi, bh, dot=None):
    dot = dot or _dot
    gi = dot(x, Wi_t) + bi
    gh = dot(h, Wh_t) + bh
    r = jax.nn.sigmoid(gi[:, :FP] + gh[:, :FP])
    z = jax.nn.sigmoid(gi[:, FP:2 * FP] + gh[:, FP:2 * FP])
    n = jnp.tanh(gi[:, 2 * FP:] + r * gh[:, 2 * FP:])
    return (1.0 - z) * n + z * h


def _body(atom_ref, bond_ref, adeg_ref, bdeg_ref, amask_ref,
          Wa_t, ba, WnA_t, WnB_t, bn,
          cat0, catR, bal, bt, Wi_t, Wh_t, bi, bh,
          wma_x_rep, wma_m_rep, bma, Wmt_t, bmt, mWi_t, mWh_t, mbi, mbh,
          wo, bo,
          af_out_ref, pred_out_ref):
    atom = atom_ref[...].reshape(BM * M, FA)
    bond = bond_ref[...].reshape(BM * M, FB)
    adeg = adeg_ref[...]
    bdeg = bdeg_ref[...]
    amask = amask_ref[...].reshape(BM * M, 1)

    col = jax.lax.broadcasted_iota(jnp.int32, (MD, M), 1)
    ohA = [(adeg[i] == col).astype(jnp.float32) for i in range(BM)]
    ohB = [(bdeg[i] == col).astype(jnp.float32) for i in range(BM)]
    seg = jax.lax.broadcasted_iota(jnp.int32, (M, MD), 1) // D
    S = (seg == jax.lax.broadcasted_iota(jnp.int32, (M, MD), 0)).astype(jnp.float32)

    adeg_f = adeg.reshape(BM * MD, 1)
    smask = jnp.where(jnp.broadcast_to(adeg_f, (BM * MD, FP)) == M - 1, -9e8, 0.0)

    atom_feat = _lrelu(_dot(atom, Wa_t[...]) + ba[...])
    P = _dot(atom, WnA_t[...])
    Q = _dot(bond, WnB_t[...])
    nf = _lrelu(jnp.concatenate(
        [_dot_x(ohA[i], P[i * M:(i + 1) * M]) + _dot_x(ohB[i], Q[i * M:(i + 1) * M])
         for i in range(BM)], axis=0) + bn[...])

    h = atom_feat
    for r in range(RADIUS):
        if r == 0:
            gath = _dot(nf, cat0[...])
            sa = _dot(atom_feat, catR[0, :, 2 * FP:])
        else:
            af = jnp.maximum(h, 0.0)
            af_ext = _dot(af, catR[r])
            gath = jnp.concatenate(
                [_dot_x(ohA[i], af_ext[i * M:(i + 1) * M, :2 * FP]) for i in range(BM)],
                axis=0)
            sa = af_ext[:, 2 * FP:]
        att = gath[:, :FP] + bt[r]
        sn = gath[:, FP:]
        score = (sn.reshape(BM * M, D, FP) + sa[:, None, :]).reshape(BM * MD, FP)
        score = _lrelu(score + bal[r]) + smask
        e = jnp.exp(score)
        X = jnp.concatenate([e * att, e], axis=1)
        nd = jnp.concatenate(
            [_dot_x(S, X[i * MD:(i + 1) * MD]) for i in range(BM)], axis=0)
        ctx = _elu(nd[:, :FP] / jnp.maximum(nd[:, FP:], 1e-30))
        x_h = atom_feat if r == 0 else h
        h = _gru(ctx, x_h, Wi_t[r], Wh_t[r], bi[r], bh[r])

    af_out_ref[...] = h.reshape(BM, M, FP)
    af_fin = jnp.maximum(h, 0.0)

    amask64 = jnp.broadcast_to(amask, (BM * M, FP))
    msm = jnp.where(amask64 == 0.0, -9e8, 0.0)
    mseg = jax.lax.broadcasted_iota(jnp.int32, (BM, BM * M), 1) // M
    onesb = (mseg == jax.lax.broadcasted_iota(jnp.int32, (BM, BM * M), 0)).astype(jnp.float32)

    mol = _dot_x(onesb, af_fin * amask64)
    afm = jnp.maximum(mol, 0.0)
    attm = _dot(af_fin, Wmt_t[...]) + bmt[...]
    sx = _dot(af_fin, wma_x_rep[...])
    for _t in range(T_STEPS):
        sm = _dot(afm, wma_m_rep[...])
        score = (sx.reshape(BM, M, FP) + sm[:, None, :]).reshape(BM * M, FP)
        score = _lrelu(score + bma[...]) + msm
        e = jnp.exp(score)
        nd = _dot_x(onesb, jnp.concatenate([e * attm, e], axis=1))
        mctx = _elu(nd[:, :FP] / jnp.maximum(nd[:, FP:], 1e-30))
        mol = _gru(mctx, mol, mWi_t[...], mWh_t[...], mbi[...], mbh[...])
        afm = jnp.maximum(mol, 0.0)

    pred = _dot(mol, wo[...]) + bo[...]
    pred_out_ref[...] = jnp.broadcast_to(pred.reshape(1, BM, 1), (1, BM, 128))


def kernel(atom_list, bond_list, atom_degree_list, bond_degree_list, atom_mask, params):
    p = params
    Wa, ba = p['atom_fc']
    Wn, bn = p['neighbor_fc']
    Wma, bma = p['mol_align']
    Wmt, bmt = p['mol_attend']
    mWi, mWh, mbi, mbh = p['mol_gru']
    Wo, bo = p['output']

    def rep(v):
        return jnp.broadcast_to(v[:, None], (FP, FP))

    cat0 = jnp.concatenate([p['attend'][0][0].T, rep(p['align'][0][0][0, FP:])], axis=1)
    catR = jnp.stack([
        jnp.concatenate([p['attend'][r][0].T,
                         rep(p['align'][r][0][0, FP:]),
                         rep(p['align'][r][0][0, :FP])], axis=1)
        for r in range(RADIUS)])
    bal = jnp.stack([p['align'][r][1] for r in range(RADIUS)])[:, :, None]
    bt = jnp.stack([p['attend'][r][1] for r in range(RADIUS)])[:, None, :]
    Wi_t = jnp.stack([p['gru'][r][0].T for r in range(RADIUS)])
    Wh_t = jnp.stack([p['gru'][r][1].T for r in range(RADIUS)])
    bi = jnp.stack([p['gru'][r][2] for r in range(RADIUS)])[:, None, :]
    bh = jnp.stack([p['gru'][r][3] for r in range(RADIUS)])[:, None, :]

    adeg = atom_degree_list.astype(jnp.int32).reshape(B, MD, 1)
    bdeg = bond_degree_list.astype(jnp.int32).reshape(B, MD, 1)
    amask = atom_mask.reshape(B, M, 1)

    data_in = [atom_list, bond_list, adeg, bdeg, amask]
    data_specs = [
        pl.BlockSpec((BM, M, FA), lambda i: (i, 0, 0)),
        pl.BlockSpec((BM, M, FB), lambda i: (i, 0, 0)),
        pl.BlockSpec((BM, MD, 1), lambda i: (i, 0, 0)),
        pl.BlockSpec((BM, MD, 1), lambda i: (i, 0, 0)),
        pl.BlockSpec((BM, M, 1), lambda i: (i, 0, 0)),
    ]
    param_in = [
        Wa.T, ba[None], Wn[:, :FA].T, Wn[:, FA:].T, bn[None],
        cat0, catR, bal, bt, Wi_t, Wh_t, bi, bh,
        rep(Wma[0, FP:]), rep(Wma[0, :FP]), bma[None], Wmt.T, bmt[None],
        mWi.T, mWh.T, mbi[None], mbh[None],
        Wo.T, bo[None],
    ]
    param_specs = [
        pl.BlockSpec(x.shape, functools.partial(lambda n, i: (0,) * n, x.ndim))
        for x in param_in
    ]

    out_shapes = [
        jax.ShapeDtypeStruct((B, M, FP), jnp.float32),
        jax.ShapeDtypeStruct((B // BM, BM, 128), jnp.float32),
    ]
    out_specs = [
        pl.BlockSpec((BM, M, FP), lambda i: (i, 0, 0)),
        pl.BlockSpec((1, BM, 128), lambda i: (i, 0, 0)),
    ]

    af, pred = pl.pallas_call(
        _body,
        grid=(B // BM,),
        in_specs=data_specs + param_specs,
        out_specs=out_specs,
        out_shape=out_shapes,
        compiler_params=pltpu.CompilerParams(
            dimension_semantics=("arbitrary",),
        ),
    )(*data_in, *param_in)
    return (af, pred.reshape(B, 128)[:, :1])
